# baseline (speedup 1.0000x reference)
import functools

import jax
import jax.numpy as jnp
from jax import lax
from jax.experimental import pallas as pl
from jax.experimental.pallas import tpu as pltpu
from jax.experimental.pallas import tpu_sc as plsc

_N0, _N1, _N2, _N3 = 50000, 10000, 10000, 10000
_D = 128
_QW = 32
_BLK = 128
_EPAD1 = 409600
_EPAD12 = 163840
_GRP = 8
_NBUF = 3
def _nacc(n):
    return ((n + 1 + 127) // 128) * 128


_ZROWS = _nacc(_N0) // 16


def _mesh():
    return plsc.VectorSubcoreMesh(core_axis_name="c", subcore_axis_name="s",
                                  num_cores=2, num_subcores=16)


@functools.lru_cache(None)
def _make_stage(n_src, n_dst, e_pad, with_ew):
    nacc = _nacc(n_dst)
    zrpt = nacc // 16
    epw = e_pad // 16
    nblk = epw // _BLK
    ngrp = nblk // _GRP
    assert nblk % _GRP == 0

    nring = 3
    ering = 3
    lead = 3

    def body(*refs):
        it = iter(refs)
        table, idx_g, idx_s = next(it), next(it), next(it)
        ewb = next(it) if with_ew else None
        zeros, out = next(it), next(it)
        idxg_st, idxs_st = next(it), next(it)
        rows = [next(it) for _ in range(nring)]
        ews = [next(it) for _ in range(ering)] if with_ew else None
        acc = next(it)
        sem_g = [next(it) for _ in range(nring)]
        c = lax.axis_index("c")
        s = lax.axis_index("s")

        def fire_g(blk, base):
            pltpu.async_copy(table.at[idxg_st.at[blk]], rows[blk % nring],
                             sem_g[blk % nring])
            if with_ew:
                pltpu.async_copy(ewb.at[pl.ds(base, _BLK), :],
                                 ews[blk % ering], sem_g[blk % nring])

        def drain_g(blk):
            pltpu.make_async_copy(table.at[pl.ds(0, _BLK)],
                                  rows[blk % nring],
                                  sem_g[blk % nring]).wait()
            if with_ew:
                pltpu.make_async_copy(ewb.at[pl.ds(0, _BLK), :],
                                      ews[blk % ering],
                                      sem_g[blk % nring]).wait()

        for j in range(2):
            q = 2 * c + j
            pltpu.sync_copy(zeros.at[pl.ds(0, zrpt)],
                            acc.at[pl.ds(s * zrpt, zrpt)])
            plsc.subcore_barrier()
            off = q * n_src

            def group(g, carry):
                grow = s * nblk + g * _GRP
                gbase = grow * _BLK
                pltpu.sync_copy(idx_g.at[pl.ds(grow, _GRP)], idxg_st)
                pltpu.sync_copy(idx_s.at[pl.ds(grow, _GRP)], idxs_st)

                def addoff(r, carry2):
                    for k in range(_BLK // 16):
                        sl = pl.ds(k * 16, 16)
                        idxg_st[r, sl] = idxg_st[r, sl] + off
                    return carry2

                lax.fori_loop(0, _GRP, addoff, 0)
                for u in range(lead):
                    fire_g(u, gbase + u * _BLK)
                for u in range(_GRP):
                    buf = u % nring
                    drain_g(u)
                    if with_ew:

                        def scale(r, carry2, _b=buf, _e=u % ering):
                            rv, ev = rows[_b], ews[_e]
                            for rr in range(8):
                                w = ev[r * 8 + rr, :]
                                for k in range(_QW // 16):
                                    sl = pl.ds(k * 16, 16)
                                    rv[r * 8 + rr, sl] = rv[r * 8 + rr, sl] * w
                            return carry2

                        lax.fori_loop(0, _BLK // 8, scale, 0)
                    pltpu.sync_copy(rows[buf], acc.at[idxs_st.at[u]],
                                    add=True)
                    if u + lead < _GRP:
                        fire_g(u + lead, gbase + (u + lead) * _BLK)
                return carry

            lax.fori_loop(0, ngrp, group, 0)
            plsc.subcore_barrier()
            pltpu.sync_copy(acc.at[pl.ds(s * zrpt, zrpt)],
                            out.at[q, pl.ds(s * zrpt, zrpt)])
            plsc.subcore_barrier()

    scratch = [
        pltpu.VMEM((_GRP, _BLK), jnp.int32),
        pltpu.VMEM((_GRP, _BLK), jnp.int32),
    ]
    scratch += [pltpu.VMEM((_BLK, _QW), jnp.float32)
                for _ in range(nring)]
    if with_ew:
        scratch += [pltpu.VMEM((_BLK, 16), jnp.float32)
                    for _ in range(ering)]
    scratch += [pltpu.VMEM_SHARED((nacc, _QW), jnp.float32)]
    scratch += [pltpu.SemaphoreType.DMA for _ in range(nring)]
    out_type = jax.ShapeDtypeStruct((4, nacc, _QW), jnp.float32)
    return pl.kernel(body, out_type=out_type, mesh=_mesh(),
                     scratch_types=scratch,
                     compiler_params=pltpu.CompilerParams(
                         use_tc_tiling_on_sc=False))


_COUNT_SPECS = (
    (_EPAD1, _N1), (_EPAD1, _N0),
    (_EPAD1, _N2), (_EPAD1, _N0),
    (_EPAD1, _N3), (_EPAD1, _N0),
    (_EPAD12, _N2), (_EPAD12, _N1),
    (_EPAD12, _N3), (_EPAD12, _N1),
)


@functools.lru_cache(None)
def _make_counts():
    nspec = len(_COUNT_SPECS)

    cgrp = 4

    def body(*refs):
        idxs = refs[:nspec]
        zeros16 = refs[nspec]
        outs = refs[nspec + 1:nspec + 1 + nspec]
        idxs_st, ones_v, acc, sem_sc = refs[nspec + 1 + nspec:]
        c = lax.axis_index("c")
        s = lax.axis_index("s")

        def ofill(r, carry):
            ones_v[r, :] = jnp.ones((16,), jnp.float32)
            return carry

        lax.fori_loop(0, _BLK, ofill, 0)
        for i, (e_pad, n_dst) in enumerate(_COUNT_SPECS):
            nacc = _nacc(n_dst)
            zrpt = nacc // 16
            nblk = e_pad // 32 // _BLK
            ngrp = nblk // cgrp
            pltpu.sync_copy(zeros16.at[pl.ds(0, zrpt)],
                            acc.at[pl.ds(s * zrpt, zrpt)])
            plsc.subcore_barrier()

            def group(g, carry, _i=i, _nblk=nblk, _epad=e_pad):
                grow = c * (_epad // 256) + s * _nblk + g * cgrp
                pltpu.sync_copy(idxs[_i].at[pl.ds(grow, cgrp)], idxs_st)
                for u in range(cgrp):
                    pltpu.async_copy(ones_v, acc.at[idxs_st.at[u]],
                                     sem_sc, add=True)
                for u in range(cgrp):
                    pltpu.make_async_copy(ones_v, acc.at[idxs_st.at[0]],
                                          sem_sc).wait()
                return carry

            lax.fori_loop(0, ngrp, group, 0)
            plsc.subcore_barrier()
            pltpu.sync_copy(acc.at[pl.ds(s * zrpt, zrpt)],
                            outs[i].at[c, pl.ds(s * zrpt, zrpt)])
            plsc.subcore_barrier()

    scratch = [
        pltpu.VMEM((cgrp, _BLK), jnp.int32),
        pltpu.VMEM((_BLK, 16), jnp.float32),
        pltpu.VMEM_SHARED((_nacc(_N0), 16), jnp.float32),
        pltpu.SemaphoreType.DMA,
    ]
    out_type = tuple(jax.ShapeDtypeStruct((2, _nacc(n), 16), jnp.float32)
                     for _, n in _COUNT_SPECS)
    return pl.kernel(body, out_type=out_type, mesh=_mesh(),
                     scratch_types=scratch,
                     compiler_params=pltpu.CompilerParams(
                         use_tc_tiling_on_sc=False))


@functools.lru_cache(None)
def _make_combine(n):
    bsz = 2000

    def body(s_ref, c_ref, x_ref, o_ref):
        cnt = jnp.maximum(c_ref[0, :, :1] + c_ref[1, :, :1], 1.0)
        o_ref[0] = (s_ref[0] / cnt + x_ref[0]) * 0.5

    grid = (4, n // bsz)
    return pl.pallas_call(
        body,
        grid=grid,
        in_specs=[
            pl.BlockSpec((1, bsz, _QW), lambda q, i: (q, i, 0)),
            pl.BlockSpec((2, bsz, 16), lambda q, i: (0, i, 0)),
            pl.BlockSpec((1, bsz, _QW), lambda q, i: (q, i, 0)),
        ],
        out_specs=pl.BlockSpec((1, bsz, _QW), lambda q, i: (q, i, 0)),
        out_shape=jax.ShapeDtypeStruct((4, n, _QW), jnp.float32),
    )


@functools.lru_cache(None)
def _make_final():
    bsz = 1000

    def body(s1, s2, s3, s4, s5, c1, c2, c3, w_ref, b_ref, att_ref, o_ref):
        srefs = (s1, s2, s3, s4, s5)
        crefs = (c1, c2, c3, c1, c1)
        hs = []
        scores = []
        for p in range(5):
            raw = jnp.concatenate([srefs[p][q] for q in range(4)], axis=1)
            cnt = jnp.maximum(crefs[p][0, :, :1] + crefs[p][1, :, :1], 1.0)
            h = raw / cnt
            h = lax.dot_general(h, w_ref[p], (((1,), (1,)), ((), ())),
                                preferred_element_type=jnp.float32)
            h = jnp.maximum(h + b_ref[p][None, :], 0.0)
            hs.append(h)
            scores.append(jnp.sum(h * att_ref[p][None, :], axis=1,
                                  keepdims=True))
        sc = jnp.concatenate(scores, axis=1)
        wts = jax.nn.softmax(sc, axis=1)
        o_ref[...] = sum(wts[:, p:p + 1] * hs[p] for p in range(5))

    spath = pl.BlockSpec((4, bsz, _QW), lambda i: (0, i, 0))
    scnt = pl.BlockSpec((2, bsz, 16), lambda i: (0, i, 0))
    return pl.pallas_call(
        body,
        grid=(_N0 // bsz,),
        in_specs=[spath] * 5 + [scnt] * 3 + [
            pl.BlockSpec((5, _D, _D), lambda i: (0, 0, 0)),
            pl.BlockSpec((5, _D), lambda i: (0, 0)),
            pl.BlockSpec((5, _D), lambda i: (0, 0)),
        ],
        out_specs=pl.BlockSpec((bsz, _D), lambda i: (i, 0)),
        out_shape=jax.ShapeDtypeStruct((_N0, _D), jnp.float32),
    )


def _stack_quarters(x):
    n = x.shape[0]
    return x.reshape(n, 4, _QW).transpose(1, 0, 2)


def _pad_idx(a, e_pad, fill):
    return jnp.pad(a, (0, e_pad - a.shape[0]), constant_values=fill)


def _pad_ewb(ew, e_pad):
    b = jnp.broadcast_to(ew[:, None], (ew.shape[0], 16))
    return jnp.pad(b, ((0, e_pad - ew.shape[0]), (0, 0)))


def kernel(x_node, x1, x2, x3, ew1, ew2, ew3, W1, b1, W2, b2, W3, b3,
           W4, b4, W5, b5, att_vec, edge_index_s1, edge_index_s2,
           edge_index_s3, edge_index_12, edge_index_13):
    xn_st = _stack_quarters(x_node)
    x1_st = _stack_quarters(x1)
    x2_st = _stack_quarters(x2)
    x3_st = _stack_quarters(x3)
    xn_tab = xn_st.reshape(4 * _N0, _QW)

    e1g = _pad_idx(edge_index_s1[0], _EPAD1, 0)
    e1s0 = _pad_idx(edge_index_s1[0], _EPAD1, _N0)
    e1d_g = _pad_idx(edge_index_s1[1], _EPAD1, 0)
    e1d_s = _pad_idx(edge_index_s1[1], _EPAD1, _N1)
    e2g = _pad_idx(edge_index_s2[0], _EPAD1, 0)
    e2s0 = _pad_idx(edge_index_s2[0], _EPAD1, _N0)
    e2d_g = _pad_idx(edge_index_s2[1], _EPAD1, 0)
    e2d_s = _pad_idx(edge_index_s2[1], _EPAD1, _N2)
    e3g = _pad_idx(edge_index_s3[0], _EPAD1, 0)
    e3s0 = _pad_idx(edge_index_s3[0], _EPAD1, _N0)
    e3d_g = _pad_idx(edge_index_s3[1], _EPAD1, 0)
    e3d_s = _pad_idx(edge_index_s3[1], _EPAD1, _N3)
    e12a_g = _pad_idx(edge_index_12[0], _EPAD12, 0)
    e12a_s = _pad_idx(edge_index_12[0], _EPAD12, _N1)
    e12b_g = _pad_idx(edge_index_12[1], _EPAD12, 0)
    e12b_s = _pad_idx(edge_index_12[1], _EPAD12, _N2)
    e13a_g = _pad_idx(edge_index_13[0], _EPAD12, 0)
    e13a_s = _pad_idx(edge_index_13[0], _EPAD12, _N1)
    e13b_g = _pad_idx(edge_index_13[1], _EPAD12, 0)
    e13b_s = _pad_idx(edge_index_13[1], _EPAD12, _N3)

    ew1b = _pad_ewb(ew1, _EPAD1)
    ew2b = _pad_ewb(ew2, _EPAD1)
    ew3b = _pad_ewb(ew3, _EPAD1)

    zeros32 = jnp.zeros((_ZROWS, _QW), jnp.float32)
    zeros16 = jnp.zeros((_ZROWS, 16), jnp.float32)

    def _rows2d(a):
        return a.reshape(-1, _BLK)

    (cnt_1d, cnt_1s, cnt_2d, cnt_2s, cnt_3d, cnt_3s,
     cnt_12b, cnt_12a, cnt_13b, cnt_13a) = _make_counts()(
        _rows2d(e1d_s), _rows2d(e1s0), _rows2d(e2d_s), _rows2d(e2s0),
        _rows2d(e3d_s), _rows2d(e3s0), _rows2d(e12b_s), _rows2d(e12a_s),
        _rows2d(e13b_s), _rows2d(e13a_s), zeros16)

    stage_h1 = _make_stage(_N0, _N1, _EPAD1, True)
    stage_h2 = _make_stage(_N1, _N0, _EPAD1, False)
    stage_mid = _make_stage(_N1, _N1, _EPAD12, False)
    stage_fin = _make_stage(_N1, _N0, _EPAD1, True)

    combine = _make_combine(_N1)

    a1 = stage_h1(xn_tab, _rows2d(e1g), _rows2d(e1d_s), ew1b, zeros32)
    net1 = combine(a1, cnt_1d, x1_st)
    net1_tab = net1.reshape(4 * _N1, _QW)
    b1s = stage_h2(net1_tab, _rows2d(e1d_g), _rows2d(e1s0), zeros32)

    a2 = stage_h1(xn_tab, _rows2d(e2g), _rows2d(e2d_s), ew2b, zeros32)
    net2 = combine(a2, cnt_2d, x2_st)
    b2s = stage_h2(net2.reshape(4 * _N2, _QW), _rows2d(e2d_g), _rows2d(e2s0), zeros32)

    a3 = stage_h1(xn_tab, _rows2d(e3g), _rows2d(e3d_s), ew3b, zeros32)
    net3 = combine(a3, cnt_3d, x3_st)
    b3s = stage_h2(net3.reshape(4 * _N3, _QW), _rows2d(e3d_g), _rows2d(e3s0), zeros32)

    m2 = stage_mid(net1_tab, _rows2d(e12a_g), _rows2d(e12b_s), zeros32)
    n2t = combine(m2, cnt_12b, x2_st)
    m3 = stage_mid(n2t.reshape(4 * _N2, _QW), _rows2d(e12b_g), _rows2d(e12a_s),
                   zeros32)
    n3t = combine(m3, cnt_12a, x1_st)
    c3s = stage_fin(n3t.reshape(4 * _N1, _QW), _rows2d(e1d_g), _rows2d(e1s0), ew1b,
                    zeros32)

    p2 = stage_mid(net1_tab, _rows2d(e13a_g), _rows2d(e13b_s), zeros32)
    q2t = combine(p2, cnt_13b, x3_st)
    p3 = stage_mid(q2t.reshape(4 * _N3, _QW), _rows2d(e13b_g), _rows2d(e13a_s),
                   zeros32)
    q3t = combine(p3, cnt_13a, x1_st)
    d3s = stage_fin(q3t.reshape(4 * _N1, _QW), _rows2d(e1d_g), _rows2d(e1s0), ew1b,
                    zeros32)

    w_stk = jnp.stack([W1, W2, W3, W4, W5])
    b_stk = jnp.stack([b1, b2, b3, b4, b5])
    return _make_final()(b1s, b2s, b3s, c3s, d3s,
                         cnt_1s, cnt_2s, cnt_3s, w_stk, b_stk, att_vec)


# speedup vs baseline: 1.0692x; 1.0692x over previous
import functools

import jax
import jax.numpy as jnp
from jax import lax
from jax.experimental import pallas as pl
from jax.experimental.pallas import tpu as pltpu
from jax.experimental.pallas import tpu_sc as plsc

_N0, _N1, _N2, _N3 = 50000, 10000, 10000, 10000
_D = 128
_QW = 32
_BLK = 128
_EPAD1 = 409600
_EPAD12 = 163840
_GRP = 8
_NBUF = 3
def _nacc(n):
    return ((n + 1 + 127) // 128) * 128


_ZROWS = _nacc(_N0) // 16


def _mesh():
    return plsc.VectorSubcoreMesh(core_axis_name="c", subcore_axis_name="s",
                                  num_cores=2, num_subcores=16)


@functools.lru_cache(None)
def _make_stage(n_src, n_dst, e_pad, with_ew):
    nacc = _nacc(n_dst)
    zrpt = nacc // 16
    epw = e_pad // 16
    nblk = epw // _BLK
    ngrp = nblk // _GRP
    assert nblk % _GRP == 0

    nring = 3
    ering = 3
    lead = 3

    def body(*refs):
        it = iter(refs)
        table, idx_g, idx_s = next(it), next(it), next(it)
        ewb = next(it) if with_ew else None
        zeros, out = next(it), next(it)
        idxg_st, idxs_st = next(it), next(it)
        rows = [next(it) for _ in range(nring)]
        ews = [next(it) for _ in range(ering)] if with_ew else None
        acc = next(it)
        sem_g = [next(it) for _ in range(nring)]
        c = lax.axis_index("c")
        s = lax.axis_index("s")

        def fire_g(blk, base):
            pltpu.async_copy(table.at[idxg_st.at[blk]], rows[blk % nring],
                             sem_g[blk % nring])
            if with_ew:
                pltpu.async_copy(ewb.at[pl.ds(base, _BLK), :],
                                 ews[blk % ering], sem_g[blk % nring])

        def drain_g(blk):
            pltpu.make_async_copy(table.at[pl.ds(0, _BLK)],
                                  rows[blk % nring],
                                  sem_g[blk % nring]).wait()
            if with_ew:
                pltpu.make_async_copy(ewb.at[pl.ds(0, _BLK), :],
                                      ews[blk % ering],
                                      sem_g[blk % nring]).wait()

        for j in range(2):
            q = 2 * c + j
            pltpu.sync_copy(zeros.at[pl.ds(0, zrpt)],
                            acc.at[pl.ds(s * zrpt, zrpt)])
            plsc.subcore_barrier()
            off = q * n_src

            def group(g, carry):
                grow = s * nblk + g * _GRP
                gbase = grow * _BLK
                pltpu.sync_copy(idx_g.at[pl.ds(grow, _GRP)], idxg_st)
                pltpu.sync_copy(idx_s.at[pl.ds(grow, _GRP)], idxs_st)

                def addoff(r, carry2):
                    for k in range(_BLK // 16):
                        sl = pl.ds(k * 16, 16)
                        idxg_st[r, sl] = idxg_st[r, sl] + off
                    return carry2

                lax.fori_loop(0, _GRP, addoff, 0)
                for u in range(lead):
                    fire_g(u, gbase + u * _BLK)
                for u in range(_GRP):
                    buf = u % nring
                    drain_g(u)
                    if with_ew:

                        def scale(r, carry2, _b=buf, _e=u % ering):
                            rv, ev = rows[_b], ews[_e]
                            for rr in range(8):
                                w = ev[r * 8 + rr, pl.ds(0, 16)]
                                for k in range(_QW // 16):
                                    sl = pl.ds(k * 16, 16)
                                    rv[r * 8 + rr, sl] = rv[r * 8 + rr, sl] * w
                            return carry2

                        lax.fori_loop(0, _BLK // 8, scale, 0)
                    pltpu.sync_copy(rows[buf], acc.at[idxs_st.at[u]],
                                    add=True)
                    if u + lead < _GRP:
                        fire_g(u + lead, gbase + (u + lead) * _BLK)
                return carry

            lax.fori_loop(0, ngrp, group, 0)
            plsc.subcore_barrier()
            pltpu.sync_copy(acc.at[pl.ds(s * zrpt, zrpt)],
                            out.at[q, pl.ds(s * zrpt, zrpt)])
            plsc.subcore_barrier()

    scratch = [
        pltpu.VMEM((_GRP, _BLK), jnp.int32),
        pltpu.VMEM((_GRP, _BLK), jnp.int32),
    ]
    scratch += [pltpu.VMEM((_BLK, _QW), jnp.float32)
                for _ in range(nring)]
    if with_ew:
        scratch += [pltpu.VMEM((_BLK, _QW), jnp.float32)
                    for _ in range(ering)]
    scratch += [pltpu.VMEM_SHARED((nacc, _QW), jnp.float32)]
    scratch += [pltpu.SemaphoreType.DMA for _ in range(nring)]
    out_type = jax.ShapeDtypeStruct((4, nacc, _QW), jnp.float32)
    return pl.kernel(body, out_type=out_type, mesh=_mesh(),
                     scratch_types=scratch,
                     compiler_params=pltpu.CompilerParams(
                         use_tc_tiling_on_sc=False))


_COUNT_SPECS = (
    (_EPAD1, _N1), (_EPAD1, _N0),
    (_EPAD1, _N2), (_EPAD1, _N0),
    (_EPAD1, _N3), (_EPAD1, _N0),
    (_EPAD12, _N2), (_EPAD12, _N1),
    (_EPAD12, _N3), (_EPAD12, _N1),
)


@functools.lru_cache(None)
def _make_counts():
    nspec = len(_COUNT_SPECS)

    cgrp = 4

    def body(*refs):
        idxs = refs[:nspec]
        zeros16 = refs[nspec]
        outs = refs[nspec + 1:nspec + 1 + nspec]
        idxs_st, ones_v, acc, sem_sc = refs[nspec + 1 + nspec:]
        c = lax.axis_index("c")
        s = lax.axis_index("s")

        def ofill(r, carry):
            ones_v[r, :] = jnp.ones((16,), jnp.float32)
            return carry

        lax.fori_loop(0, _BLK, ofill, 0)
        for i, (e_pad, n_dst) in enumerate(_COUNT_SPECS):
            nacc = _nacc(n_dst)
            zrpt = nacc // 16
            nblk = e_pad // 32 // _BLK
            ngrp = nblk // cgrp
            pltpu.sync_copy(zeros16.at[pl.ds(0, zrpt)],
                            acc.at[pl.ds(s * zrpt, zrpt)])
            plsc.subcore_barrier()

            def group(g, carry, _i=i, _nblk=nblk, _epad=e_pad):
                grow = c * (_epad // 256) + s * _nblk + g * cgrp
                pltpu.sync_copy(idxs[_i].at[pl.ds(grow, cgrp)], idxs_st)
                for u in range(cgrp):
                    pltpu.async_copy(ones_v, acc.at[idxs_st.at[u]],
                                     sem_sc, add=True)
                for u in range(cgrp):
                    pltpu.make_async_copy(ones_v, acc.at[idxs_st.at[0]],
                                          sem_sc).wait()
                return carry

            lax.fori_loop(0, ngrp, group, 0)
            plsc.subcore_barrier()
            pltpu.sync_copy(acc.at[pl.ds(s * zrpt, zrpt)],
                            outs[i].at[c, pl.ds(s * zrpt, zrpt)])
            plsc.subcore_barrier()

    scratch = [
        pltpu.VMEM((cgrp, _BLK), jnp.int32),
        pltpu.VMEM((_BLK, 16), jnp.float32),
        pltpu.VMEM_SHARED((_nacc(_N0), 16), jnp.float32),
        pltpu.SemaphoreType.DMA,
    ]
    out_type = tuple(jax.ShapeDtypeStruct((2, _nacc(n), 16), jnp.float32)
                     for _, n in _COUNT_SPECS)
    return pl.kernel(body, out_type=out_type, mesh=_mesh(),
                     scratch_types=scratch,
                     compiler_params=pltpu.CompilerParams(
                         use_tc_tiling_on_sc=False))


@functools.lru_cache(None)
def _make_combine(n):
    bsz = 2000

    def body(s_ref, c_ref, x_ref, o_ref):
        cnt = jnp.maximum(c_ref[0, :, :1] + c_ref[1, :, :1], 1.0)
        o_ref[0] = (s_ref[0] / cnt + x_ref[0]) * 0.5

    grid = (4, n // bsz)
    return pl.pallas_call(
        body,
        grid=grid,
        in_specs=[
            pl.BlockSpec((1, bsz, _QW), lambda q, i: (q, i, 0)),
            pl.BlockSpec((2, bsz, 16), lambda q, i: (0, i, 0)),
            pl.BlockSpec((1, bsz, _QW), lambda q, i: (q, i, 0)),
        ],
        out_specs=pl.BlockSpec((1, bsz, _QW), lambda q, i: (q, i, 0)),
        out_shape=jax.ShapeDtypeStruct((4, n, _QW), jnp.float32),
    )


@functools.lru_cache(None)
def _make_final():
    bsz = 1000

    def body(s1, s2, s3, s4, s5, c1, c2, c3, w_ref, b_ref, att_ref, o_ref):
        srefs = (s1, s2, s3, s4, s5)
        crefs = (c1, c2, c3, c1, c1)
        hs = []
        scores = []
        for p in range(5):
            raw = jnp.concatenate([srefs[p][q] for q in range(4)], axis=1)
            cnt = jnp.maximum(crefs[p][0, :, :1] + crefs[p][1, :, :1], 1.0)
            h = raw / cnt
            h = lax.dot_general(h, w_ref[p], (((1,), (1,)), ((), ())),
                                preferred_element_type=jnp.float32)
            h = jnp.maximum(h + b_ref[p][None, :], 0.0)
            hs.append(h)
            scores.append(jnp.sum(h * att_ref[p][None, :], axis=1,
                                  keepdims=True))
        sc = jnp.concatenate(scores, axis=1)
        wts = jax.nn.softmax(sc, axis=1)
        o_ref[...] = sum(wts[:, p:p + 1] * hs[p] for p in range(5))

    spath = pl.BlockSpec((4, bsz, _QW), lambda i: (0, i, 0))
    scnt = pl.BlockSpec((2, bsz, 16), lambda i: (0, i, 0))
    return pl.pallas_call(
        body,
        grid=(_N0 // bsz,),
        in_specs=[spath] * 5 + [scnt] * 3 + [
            pl.BlockSpec((5, _D, _D), lambda i: (0, 0, 0)),
            pl.BlockSpec((5, _D), lambda i: (0, 0)),
            pl.BlockSpec((5, _D), lambda i: (0, 0)),
        ],
        out_specs=pl.BlockSpec((bsz, _D), lambda i: (i, 0)),
        out_shape=jax.ShapeDtypeStruct((_N0, _D), jnp.float32),
    )


def _stack_quarters(x):
    n = x.shape[0]
    return x.reshape(n, 4, _QW).transpose(1, 0, 2)


def _pad_idx(a, e_pad, fill):
    return jnp.pad(a, (0, e_pad - a.shape[0]), constant_values=fill)


def _pad_ewb(ew, e_pad):
    b = jnp.broadcast_to(ew[:, None], (ew.shape[0], _QW))
    return jnp.pad(b, ((0, e_pad - ew.shape[0]), (0, 0)))


def kernel(x_node, x1, x2, x3, ew1, ew2, ew3, W1, b1, W2, b2, W3, b3,
           W4, b4, W5, b5, att_vec, edge_index_s1, edge_index_s2,
           edge_index_s3, edge_index_12, edge_index_13):
    xn_st = _stack_quarters(x_node)
    x1_st = _stack_quarters(x1)
    x2_st = _stack_quarters(x2)
    x3_st = _stack_quarters(x3)
    xn_tab = xn_st.reshape(4 * _N0, _QW)

    e1g = _pad_idx(edge_index_s1[0], _EPAD1, 0)
    e1s0 = _pad_idx(edge_index_s1[0], _EPAD1, _N0)
    e1d_g = _pad_idx(edge_index_s1[1], _EPAD1, 0)
    e1d_s = _pad_idx(edge_index_s1[1], _EPAD1, _N1)
    e2g = _pad_idx(edge_index_s2[0], _EPAD1, 0)
    e2s0 = _pad_idx(edge_index_s2[0], _EPAD1, _N0)
    e2d_g = _pad_idx(edge_index_s2[1], _EPAD1, 0)
    e2d_s = _pad_idx(edge_index_s2[1], _EPAD1, _N2)
    e3g = _pad_idx(edge_index_s3[0], _EPAD1, 0)
    e3s0 = _pad_idx(edge_index_s3[0], _EPAD1, _N0)
    e3d_g = _pad_idx(edge_index_s3[1], _EPAD1, 0)
    e3d_s = _pad_idx(edge_index_s3[1], _EPAD1, _N3)
    e12a_g = _pad_idx(edge_index_12[0], _EPAD12, 0)
    e12a_s = _pad_idx(edge_index_12[0], _EPAD12, _N1)
    e12b_g = _pad_idx(edge_index_12[1], _EPAD12, 0)
    e12b_s = _pad_idx(edge_index_12[1], _EPAD12, _N2)
    e13a_g = _pad_idx(edge_index_13[0], _EPAD12, 0)
    e13a_s = _pad_idx(edge_index_13[0], _EPAD12, _N1)
    e13b_g = _pad_idx(edge_index_13[1], _EPAD12, 0)
    e13b_s = _pad_idx(edge_index_13[1], _EPAD12, _N3)

    ew1b = _pad_ewb(ew1, _EPAD1)
    ew2b = _pad_ewb(ew2, _EPAD1)
    ew3b = _pad_ewb(ew3, _EPAD1)

    zeros32 = jnp.zeros((_ZROWS, _QW), jnp.float32)
    zeros16 = jnp.zeros((_ZROWS, 16), jnp.float32)

    def _rows2d(a):
        return a.reshape(-1, _BLK)

    (cnt_1d, cnt_1s, cnt_2d, cnt_2s, cnt_3d, cnt_3s,
     cnt_12b, cnt_12a, cnt_13b, cnt_13a) = _make_counts()(
        _rows2d(e1d_s), _rows2d(e1s0), _rows2d(e2d_s), _rows2d(e2s0),
        _rows2d(e3d_s), _rows2d(e3s0), _rows2d(e12b_s), _rows2d(e12a_s),
        _rows2d(e13b_s), _rows2d(e13a_s), zeros16)

    stage_h1 = _make_stage(_N0, _N1, _EPAD1, True)
    stage_h2 = _make_stage(_N1, _N0, _EPAD1, False)
    stage_mid = _make_stage(_N1, _N1, _EPAD12, False)
    stage_fin = _make_stage(_N1, _N0, _EPAD1, True)

    combine = _make_combine(_N1)

    a1 = stage_h1(xn_tab, _rows2d(e1g), _rows2d(e1d_s), ew1b, zeros32)
    net1 = combine(a1, cnt_1d, x1_st)
    net1_tab = net1.reshape(4 * _N1, _QW)
    b1s = stage_h2(net1_tab, _rows2d(e1d_g), _rows2d(e1s0), zeros32)

    a2 = stage_h1(xn_tab, _rows2d(e2g), _rows2d(e2d_s), ew2b, zeros32)
    net2 = combine(a2, cnt_2d, x2_st)
    b2s = stage_h2(net2.reshape(4 * _N2, _QW), _rows2d(e2d_g), _rows2d(e2s0), zeros32)

    a3 = stage_h1(xn_tab, _rows2d(e3g), _rows2d(e3d_s), ew3b, zeros32)
    net3 = combine(a3, cnt_3d, x3_st)
    b3s = stage_h2(net3.reshape(4 * _N3, _QW), _rows2d(e3d_g), _rows2d(e3s0), zeros32)

    m2 = stage_mid(net1_tab, _rows2d(e12a_g), _rows2d(e12b_s), zeros32)
    n2t = combine(m2, cnt_12b, x2_st)
    m3 = stage_mid(n2t.reshape(4 * _N2, _QW), _rows2d(e12b_g), _rows2d(e12a_s),
                   zeros32)
    n3t = combine(m3, cnt_12a, x1_st)
    c3s = stage_fin(n3t.reshape(4 * _N1, _QW), _rows2d(e1d_g), _rows2d(e1s0), ew1b,
                    zeros32)

    p2 = stage_mid(net1_tab, _rows2d(e13a_g), _rows2d(e13b_s), zeros32)
    q2t = combine(p2, cnt_13b, x3_st)
    p3 = stage_mid(q2t.reshape(4 * _N3, _QW), _rows2d(e13b_g), _rows2d(e13a_s),
                   zeros32)
    q3t = combine(p3, cnt_13a, x1_st)
    d3s = stage_fin(q3t.reshape(4 * _N1, _QW), _rows2d(e1d_g), _rows2d(e1s0), ew1b,
                    zeros32)

    w_stk = jnp.stack([W1, W2, W3, W4, W5])
    b_stk = jnp.stack([b1, b2, b3, b4, b5])
    return _make_final()(b1s, b2s, b3s, c3s, d3s,
                         cnt_1s, cnt_2s, cnt_3s, w_stk, b_stk, att_vec)
